# baseline (speedup 1.0000x reference)
import functools

import jax
import jax.numpy as jnp
from jax import lax
from jax.experimental import pallas as pl
from jax.experimental.pallas import tpu as pltpu
from jax.experimental.pallas import tpu_sc as plsc

CS = 16


@functools.lru_cache(maxsize=None)
def _make(batch, fields, D):
    info = plsc.get_sparse_core_info()
    NC, NS = info.num_cores, info.num_subcores
    NW = NC * NS
    assert batch % (NW * CS) == 0
    s_per_w = batch // NW
    n_groups = s_per_w // CS
    mesh = plsc.VectorSubcoreMesh(core_axis_name="c", subcore_axis_name="s")

    @functools.partial(
        pl.kernel,
        mesh=mesh,
        out_type=jax.ShapeDtypeStruct((batch, fields, D), jnp.float32),
        scratch_types=[
            pltpu.VMEM((s_per_w, fields), jnp.int32),
            pltpu.VMEM((2, CS, fields, D), jnp.float32),
            pltpu.SemaphoreType.DMA,
            pltpu.SemaphoreType.DMA,
        ],
        compiler_params=pltpu.CompilerParams(use_tc_tiling_on_sc=False),
    )
    def k(idx_hbm, table_hbm, out_hbm, idx_v, bufs, gsem, wsem):
        wid = lax.axis_index("s") * NC + lax.axis_index("c")
        base = wid * s_per_w
        pltpu.sync_copy(idx_hbm.at[pl.ds(base, s_per_w)], idx_v)

        def issue_gathers(g, p):
            for b in range(CS):
                pltpu.async_copy(
                    table_hbm.at[idx_v.at[g * CS + b]], bufs.at[p, b], gsem)

        def wait_gathers(p):
            for b in range(CS):
                pltpu.make_async_copy(
                    table_hbm.at[idx_v.at[0]], bufs.at[p, b], gsem).wait()

        def issue_write(g, p):
            pltpu.async_copy(
                bufs.at[p], out_hbm.at[pl.ds(base + g * CS, CS)], wsem)

        def wait_write(p):
            pltpu.make_async_copy(
                bufs.at[p], out_hbm.at[pl.ds(base, CS)], wsem).wait()

        issue_gathers(0, 0)

        def body(g, carry):
            p = lax.rem(g, 2)
            wait_gathers(p)

            @pl.when(g > 0)
            def _():
                wait_write(1 - p)

            @pl.when(g + 1 < n_groups)
            def _():
                issue_gathers(g + 1, 1 - p)

            issue_write(g, p)
            return carry

        lax.fori_loop(0, n_groups, body, 0)
        wait_write((n_groups - 1) % 2)

    return k


def kernel(index, table):
    batch, fields = index.shape
    D = table.shape[1]
    return _make(batch, fields, D)(index.astype(jnp.int32), table)


# speedup vs baseline: 1.0011x; 1.0011x over previous
import functools

import jax
import jax.numpy as jnp
from jax import lax
from jax.experimental import pallas as pl
from jax.experimental.pallas import tpu as pltpu
from jax.experimental.pallas import tpu_sc as plsc

CS = 8
PADDED = 128


@functools.lru_cache(maxsize=None)
def _make(batch, fields, D):
    info = plsc.get_sparse_core_info()
    NC, NS = info.num_cores, info.num_subcores
    NW = NC * NS
    assert batch % (NW * CS) == 0
    s_per_w = batch // NW
    n_groups = s_per_w // CS
    mesh = plsc.VectorSubcoreMesh(core_axis_name="c", subcore_axis_name="s")

    @functools.partial(
        pl.kernel,
        mesh=mesh,
        out_type=jax.ShapeDtypeStruct((batch, fields, D), jnp.float32),
        scratch_types=[
            pltpu.VMEM((s_per_w, fields), jnp.int32),
            pltpu.VMEM((2, CS, fields, PADDED), jnp.float32),
            pltpu.SemaphoreType.DMA,
            pltpu.SemaphoreType.DMA,
        ],
        compiler_params=pltpu.CompilerParams(use_tc_tiling_on_sc=False),
    )
    def k(idx_hbm, table_hbm, out_hbm, idx_v, bufs, gsem, wsem):
        wid = lax.axis_index("s") * NC + lax.axis_index("c")
        base = wid * s_per_w
        pltpu.sync_copy(idx_hbm.at[pl.ds(base, s_per_w)], idx_v)

        def issue_gathers(g, p):
            for b in range(CS):
                pltpu.async_copy(
                    table_hbm.at[idx_v.at[g * CS + b]], bufs.at[p, b], gsem)

        def wait_gathers(p):
            for b in range(CS):
                pltpu.make_async_copy(
                    table_hbm.at[idx_v.at[0]], bufs.at[p, b], gsem).wait()

        def issue_write(g, p):
            pltpu.async_copy(
                bufs.at[p, :, :, pl.ds(0, D)],
                out_hbm.at[pl.ds(base + g * CS, CS)], wsem)

        def wait_write(p):
            pltpu.make_async_copy(
                bufs.at[p, :, :, pl.ds(0, D)],
                out_hbm.at[pl.ds(base, CS)], wsem).wait()

        issue_gathers(0, 0)

        def body(g, carry):
            p = lax.rem(g, 2)
            wait_gathers(p)

            @pl.when(g > 0)
            def _():
                wait_write(1 - p)

            @pl.when(g + 1 < n_groups)
            def _():
                issue_gathers(g + 1, 1 - p)

            issue_write(g, p)
            return carry

        lax.fori_loop(0, n_groups, body, 0)
        wait_write((n_groups - 1) % 2)

    return k


def kernel(index, table):
    batch, fields = index.shape
    D = table.shape[1]
    tab = jnp.pad(table, ((0, 0), (0, PADDED - D)))
    return _make(batch, fields, D)(index.astype(jnp.int32), tab)


# speedup vs baseline: 1.0746x; 1.0734x over previous
import functools

import jax
import jax.numpy as jnp
from jax import lax
from jax.experimental import pallas as pl
from jax.experimental.pallas import tpu as pltpu
from jax.experimental.pallas import tpu_sc as plsc

CS = 16
PADDED = 128


@functools.lru_cache(maxsize=None)
def _make(batch, fields, D):
    info = plsc.get_sparse_core_info()
    NC, NS = info.num_cores, info.num_subcores
    NW = NC * NS
    assert batch % (NW * CS) == 0
    s_per_w = batch // NW
    n_groups = s_per_w // CS
    mesh = plsc.VectorSubcoreMesh(core_axis_name="c", subcore_axis_name="s")

    @functools.partial(
        pl.kernel,
        mesh=mesh,
        out_type=jax.ShapeDtypeStruct((batch, fields, D), jnp.float32),
        scratch_types=[
            pltpu.VMEM((s_per_w, fields), jnp.int32),
            pltpu.VMEM((2, CS, fields, D), jnp.float32),
            pltpu.SemaphoreType.DMA,
            pltpu.SemaphoreType.DMA,
        ],
        compiler_params=pltpu.CompilerParams(use_tc_tiling_on_sc=False),
    )
    def k(idx_hbm, table_hbm, out_hbm, idx_v, bufs, gsem, wsem):
        wid = lax.axis_index("s") * NC + lax.axis_index("c")
        base = wid * s_per_w
        pltpu.sync_copy(idx_hbm.at[pl.ds(base, s_per_w)], idx_v)

        def issue_gathers(g, p):
            for b in range(CS):
                pltpu.async_copy(
                    table_hbm.at[idx_v.at[g * CS + b]], bufs.at[p, b], gsem)

        def wait_gathers(p):
            for b in range(CS):
                pltpu.make_async_copy(
                    table_hbm.at[idx_v.at[0]], bufs.at[p, b], gsem).wait()

        def issue_write(g, p):
            pltpu.async_copy(
                bufs.at[p], out_hbm.at[pl.ds(base + g * CS, CS)], wsem)

        def wait_write(p):
            pltpu.make_async_copy(
                bufs.at[p], out_hbm.at[pl.ds(base, CS)], wsem).wait()

        issue_gathers(0, 0)

        def body(g, carry):
            p = lax.rem(g, 2)
            wait_gathers(p)

            @pl.when(g > 0)
            def _():
                wait_write(1 - p)

            @pl.when(g + 1 < n_groups)
            def _():
                issue_gathers(g + 1, 1 - p)

            issue_write(g, p)
            return carry

        lax.fori_loop(0, n_groups, body, 0)
        wait_write((n_groups - 1) % 2)

    return k


def kernel(index, table):
    batch, fields = index.shape
    V, D = table.shape
    tab2 = jnp.pad(table, ((0, 0), (0, PADDED - D)))
    tab2 = tab2.reshape(V * (PADDED // D), D)
    idx = index.astype(jnp.int32) * (PADDED // D)
    return _make(batch, fields, D)(idx, tab2)


# speedup vs baseline: 1.0794x; 1.0045x over previous
import functools

import jax
import jax.numpy as jnp
from jax import lax
from jax.experimental import pallas as pl
from jax.experimental.pallas import tpu as pltpu
from jax.experimental.pallas import tpu_sc as plsc

CS = 32
PADDED = 128


@functools.lru_cache(maxsize=None)
def _make(batch, fields, D):
    info = plsc.get_sparse_core_info()
    NC, NS = info.num_cores, info.num_subcores
    NW = NC * NS
    assert batch % (NW * CS) == 0
    s_per_w = batch // NW
    n_groups = s_per_w // CS
    mesh = plsc.VectorSubcoreMesh(core_axis_name="c", subcore_axis_name="s")

    @functools.partial(
        pl.kernel,
        mesh=mesh,
        out_type=jax.ShapeDtypeStruct((batch, fields, D), jnp.float32),
        scratch_types=[
            pltpu.VMEM((s_per_w, fields), jnp.int32),
            pltpu.VMEM((2, CS, fields, D), jnp.float32),
            pltpu.SemaphoreType.DMA,
            pltpu.SemaphoreType.DMA,
        ],
        compiler_params=pltpu.CompilerParams(use_tc_tiling_on_sc=False),
    )
    def k(idx_hbm, table_hbm, out_hbm, idx_v, bufs, gsem, wsem):
        wid = lax.axis_index("s") * NC + lax.axis_index("c")
        base = wid * s_per_w
        pltpu.sync_copy(idx_hbm.at[pl.ds(base, s_per_w)], idx_v)

        def issue_gathers(g, p):
            for b in range(CS):
                pltpu.async_copy(
                    table_hbm.at[idx_v.at[g * CS + b]], bufs.at[p, b], gsem)

        def wait_gathers(p):
            for b in range(CS):
                pltpu.make_async_copy(
                    table_hbm.at[idx_v.at[0]], bufs.at[p, b], gsem).wait()

        def issue_write(g, p):
            pltpu.async_copy(
                bufs.at[p], out_hbm.at[pl.ds(base + g * CS, CS)], wsem)

        def wait_write(p):
            pltpu.make_async_copy(
                bufs.at[p], out_hbm.at[pl.ds(base, CS)], wsem).wait()

        issue_gathers(0, 0)

        def body(g, carry):
            p = lax.rem(g, 2)
            wait_gathers(p)

            @pl.when(g > 0)
            def _():
                wait_write(1 - p)

            @pl.when(g + 1 < n_groups)
            def _():
                issue_gathers(g + 1, 1 - p)

            issue_write(g, p)
            return carry

        lax.fori_loop(0, n_groups, body, 0)
        wait_write((n_groups - 1) % 2)

    return k


def kernel(index, table):
    batch, fields = index.shape
    V, D = table.shape
    tab2 = jnp.pad(table, ((0, 0), (0, PADDED - D)))
    tab2 = tab2.reshape(V * (PADDED // D), D)
    idx = index.astype(jnp.int32) * (PADDED // D)
    return _make(batch, fields, D)(idx, tab2)
